# baseline (speedup 1.0000x reference)
import jax
import jax.numpy as jnp
from jax import lax
from jax.experimental import pallas as pl
from jax.experimental.pallas import tpu as pltpu
from jax.experimental.pallas import tpu_sc as plsc

VOCAB = 1000000
WORD_DIM = 128
BATCH = 4096
SEQ = 50

NC = 2
NS = 16
NW = NC * NS

B = BATCH * SEQ
CHUNK = 128
NS_CHUNKS = 45
B_SC = NW * NS_CHUNKS * CHUNK
B_TC = B - B_SC
NBUF = 4

IBLK = 1024
N_BLKS = B_TC // IBLK


def _sc_body(x_hbm, table_hbm, out_hbm,
             idx_v, buf0, buf1, buf2, buf3, g0, g1, g2, g3):
    bufs = (buf0, buf1, buf2, buf3)
    gsem = (g0, g1, g2, g3)
    wid = lax.axis_index("s") * NC + lax.axis_index("c")
    base = wid * NS_CHUNKS * CHUNK
    pltpu.sync_copy(x_hbm.at[wid], idx_v)

    for b in range(NBUF - 1):
        pltpu.async_copy(table_hbm.at[idx_v.at[b]], bufs[b], gsem[b])

    def step(j, _):
        for p in range(NBUF):
            @pl.when(j % NBUF == p)
            def _(p=p):
                q = (p + NBUF - 1) % NBUF

                @pl.when(j + NBUF - 1 < NS_CHUNKS)
                def _():
                    pltpu.async_copy(
                        table_hbm.at[idx_v.at[j + NBUF - 1]], bufs[q],
                        gsem[q])
                pltpu.make_async_copy(
                    table_hbm.at[idx_v.at[j]], bufs[p], gsem[p]).wait()
                pltpu.sync_copy(
                    bufs[p], out_hbm.at[pl.ds(base + j * CHUNK, CHUNK)])

        return 0

    lax.fori_loop(0, NS_CHUNKS, step, 0)


def _tc_body(idx_hbm, table_hbm, out_hbm, idx_s0, idx_s1, isem0, isem1,
             rsem0, rsem1):
    idx_s = (idx_s0, idx_s1)
    isem = (isem0, isem1)
    rsem = (rsem0, rsem1)

    pltpu.async_copy(idx_hbm.at[pl.ds(0, IBLK)], idx_s0, isem0)

    def blk(b, _):
        for p in range(2):
            @pl.when(b % 2 == p)
            def _(p=p):
                q = 1 - p

                @pl.when(b + 1 < N_BLKS)
                def _():
                    pltpu.async_copy(
                        idx_hbm.at[pl.ds((b + 1) * IBLK, IBLK)], idx_s[q],
                        isem[q])
                pltpu.make_async_copy(
                    idx_hbm.at[pl.ds(b * IBLK, IBLK)], idx_s[p],
                    isem[p]).wait()

                def row(i, _):
                    r = b * IBLK + i
                    v = idx_s[p][i]
                    pltpu.async_copy(
                        table_hbm.at[pl.ds(v, 1)],
                        out_hbm.at[pl.ds(r, 1)], rsem[p])
                    return 0

                lax.fori_loop(0, IBLK, row, 0)

                @pl.when(b >= 1)
                def _():
                    pltpu.make_async_copy(
                        table_hbm.at[pl.ds(0, IBLK)],
                        out_hbm.at[pl.ds((b - 1) * IBLK, IBLK)],
                        rsem[q]).wait()

        return 0

    lax.fori_loop(0, N_BLKS, blk, 0)
    lastp = (N_BLKS - 1) % 2
    pltpu.make_async_copy(
        table_hbm.at[pl.ds(0, IBLK)],
        out_hbm.at[pl.ds((N_BLKS - 1) * IBLK, IBLK)],
        rsem[lastp]).wait()


@jax.jit
def _embed(x_sc, x_tc, table):
    mesh = plsc.VectorSubcoreMesh(core_axis_name="c", subcore_axis_name="s")
    out_sc = pl.kernel(
        _sc_body,
        out_type=jax.ShapeDtypeStruct((B, WORD_DIM), jnp.float32),
        mesh=mesh,
        scratch_types=[
            pltpu.VMEM((NS_CHUNKS, CHUNK), jnp.int32),
            pltpu.VMEM((CHUNK, WORD_DIM), jnp.float32),
            pltpu.VMEM((CHUNK, WORD_DIM), jnp.float32),
            pltpu.VMEM((CHUNK, WORD_DIM), jnp.float32),
            pltpu.VMEM((CHUNK, WORD_DIM), jnp.float32),
            pltpu.SemaphoreType.DMA,
            pltpu.SemaphoreType.DMA,
            pltpu.SemaphoreType.DMA,
            pltpu.SemaphoreType.DMA,
        ],
    )(x_sc, table)

    out_tc = pl.pallas_call(
        _tc_body,
        out_shape=jax.ShapeDtypeStruct((B_TC, WORD_DIM), jnp.float32),
        in_specs=[
            pl.BlockSpec(memory_space=pl.ANY),
            pl.BlockSpec(memory_space=pl.ANY),
        ],
        out_specs=pl.BlockSpec(memory_space=pl.ANY),
        scratch_shapes=[
            pltpu.SMEM((IBLK,), jnp.int32),
            pltpu.SMEM((IBLK,), jnp.int32),
            pltpu.SemaphoreType.DMA,
            pltpu.SemaphoreType.DMA,
            pltpu.SemaphoreType.DMA,
            pltpu.SemaphoreType.DMA,
        ],
    )(x_tc, table)

    return lax.dynamic_update_slice(out_sc, out_tc, (B_SC, 0))


def kernel(x, lengths, table):
    x_flat = x.reshape(B)
    x_sc = x_flat[:B_SC].reshape(NW, NS_CHUNKS, CHUNK)
    x_tc = x_flat[B_SC:]
    out = _embed(x_sc, x_tc, table)
    emb = out.reshape(BATCH, SEQ, WORD_DIM)
    return (emb, lengths, emb)


# speedup vs baseline: 1.7094x; 1.7094x over previous
import jax
import jax.numpy as jnp
from jax import lax
from jax.experimental import pallas as pl
from jax.experimental.pallas import tpu as pltpu
from jax.experimental.pallas import tpu_sc as plsc

VOCAB = 1000000
WORD_DIM = 128
BATCH = 4096
SEQ = 50

NC = 2
NS = 16
NW = NC * NS

B = BATCH * SEQ
B_PER_W = B // NW
CHUNK = 128
N_CHUNKS = B_PER_W // CHUNK
NBUF = 4


def _gather_body(x_hbm, table_hbm, out_hbm,
                 idx_v, buf0, buf1, buf2, buf3, g0, g1, g2, g3):
    bufs = (buf0, buf1, buf2, buf3)
    gsem = (g0, g1, g2, g3)
    wid = lax.axis_index("s") * NC + lax.axis_index("c")
    base = wid * B_PER_W
    pltpu.sync_copy(x_hbm.at[wid], idx_v)

    for b in range(NBUF - 1):
        pltpu.async_copy(table_hbm.at[idx_v.at[b]], bufs[b], gsem[b])

    def step(j, _):
        for p in range(NBUF):
            @pl.when(j % NBUF == p)
            def _(p=p):
                q = (p + NBUF - 1) % NBUF
                @pl.when(j + NBUF - 1 < N_CHUNKS)
                def _():
                    pltpu.async_copy(
                        table_hbm.at[idx_v.at[j + NBUF - 1]], bufs[q],
                        gsem[q])
                pltpu.make_async_copy(
                    table_hbm.at[idx_v.at[j]], bufs[p], gsem[p]).wait()
                pltpu.sync_copy(
                    bufs[p], out_hbm.at[pl.ds(base + j * CHUNK, CHUNK)])

        return 0

    lax.fori_loop(0, N_CHUNKS, step, 0)


@jax.jit
def _embed(x_flat, table):
    mesh = plsc.VectorSubcoreMesh(core_axis_name="c", subcore_axis_name="s")
    run = pl.kernel(
        _gather_body,
        out_type=jax.ShapeDtypeStruct((B, WORD_DIM), jnp.float32),
        mesh=mesh,
        scratch_types=[
            pltpu.VMEM((N_CHUNKS, CHUNK), jnp.int32),
            pltpu.VMEM((CHUNK, WORD_DIM), jnp.float32),
            pltpu.VMEM((CHUNK, WORD_DIM), jnp.float32),
            pltpu.VMEM((CHUNK, WORD_DIM), jnp.float32),
            pltpu.VMEM((CHUNK, WORD_DIM), jnp.float32),
            pltpu.SemaphoreType.DMA,
            pltpu.SemaphoreType.DMA,
            pltpu.SemaphoreType.DMA,
            pltpu.SemaphoreType.DMA,
        ],
    )
    return run(x_flat, table)


def kernel(x, lengths, table):
    x_flat = x.reshape(NW, N_CHUNKS, CHUNK)
    out = _embed(x_flat, table)
    emb = out.reshape(BATCH, SEQ, WORD_DIM)
    return (emb, lengths, emb)
